# baseline (speedup 1.0000x reference)
import functools

import jax
import jax.numpy as jnp
from jax import lax
from jax.experimental import pallas as pl
from jax.experimental.pallas import tpu as pltpu
from jax.experimental.pallas import tpu_sc as plsc

D_MODEL = 64
BATCH = 4096
SEQ = 200
NUM_ROWS = BATCH * SEQ
NUM_PAIRS = NUM_ROWS // 2

NUM_CORES = 2
NUM_SUBCORES = 16
NUM_WORKERS = NUM_CORES * NUM_SUBCORES
PAIRS_PER_W = NUM_PAIRS // NUM_WORKERS
C2 = 320
NUM_CHUNKS = PAIRS_PER_W // C2

_MESH = plsc.VectorSubcoreMesh(core_axis_name="c", subcore_axis_name="s")


@functools.partial(
    pl.kernel,
    mesh=_MESH,
    out_type=jax.ShapeDtypeStruct((NUM_PAIRS, 2 * D_MODEL), jnp.float32),
    scratch_types=[
        pltpu.VMEM((PAIRS_PER_W,), jnp.int32),
        pltpu.VMEM((PAIRS_PER_W,), jnp.int32),
        pltpu.VMEM((C2, D_MODEL), jnp.float32),
        pltpu.VMEM((C2, D_MODEL), jnp.float32),
        pltpu.VMEM((C2, D_MODEL), jnp.float32),
        pltpu.VMEM((C2, D_MODEL), jnp.float32),
        pltpu.SemaphoreType.DMA,
        pltpu.SemaphoreType.DMA,
        pltpu.SemaphoreType.DMA,
        pltpu.SemaphoreType.DMA,
        pltpu.SemaphoreType.DMA,
        pltpu.SemaphoreType.DMA,
        pltpu.SemaphoreType.DMA,
        pltpu.SemaphoreType.DMA,
    ],
    compiler_params=pltpu.CompilerParams(use_tc_tiling_on_sc=False),
)
def _sc_gather(
    idx_e_hbm, idx_o_hbm, table_hbm, stag_hbm,
    idxe_v, idxo_v,
    rbe0, rbe1, rbo0, rbo1,
    ge0, ge1, go0, go1, se0, se1, so0, so1,
):
    wid = lax.axis_index("s") * NUM_CORES + lax.axis_index("c")
    base = wid * PAIRS_PER_W
    pltpu.sync_copy(idx_e_hbm.at[pl.ds(base, PAIRS_PER_W)], idxe_v)
    pltpu.sync_copy(idx_o_hbm.at[pl.ds(base, PAIRS_PER_W)], idxo_v)

    rbe = (rbe0, rbe1)
    rbo = (rbo0, rbo1)
    ge = (ge0, ge1)
    go = (go0, go1)
    se = (se0, se1)
    so = (so0, so1)

    def gather_e(i, b):
        return pltpu.make_async_copy(
            table_hbm.at[idxe_v.at[pl.ds(i * C2, C2)]], rbe[b], ge[b]
        )

    def gather_o(i, b):
        return pltpu.make_async_copy(
            table_hbm.at[idxo_v.at[pl.ds(i * C2, C2)]], rbo[b], go[b]
        )

    def store_e(i, b):
        return pltpu.make_async_copy(
            rbe[b],
            stag_hbm.at[pl.ds(base + i * C2, C2), pl.ds(0, D_MODEL)],
            se[b],
        )

    def store_o(i, b):
        return pltpu.make_async_copy(
            rbo[b],
            stag_hbm.at[pl.ds(base + i * C2, C2), pl.ds(D_MODEL, D_MODEL)],
            so[b],
        )

    gather_e(0, 0).start()
    gather_o(0, 0).start()
    for i in range(NUM_CHUNKS):
        b = i % 2
        gather_e(i, b).wait()
        gather_o(i, b).wait()
        if i >= 1:
            store_e(i - 1, 1 - b).wait()
            store_o(i - 1, 1 - b).wait()
        if i + 1 < NUM_CHUNKS:
            gather_e(i + 1, 1 - b).start()
            gather_o(i + 1, 1 - b).start()
        store_e(i, b).start()
        store_o(i, b).start()
    bl = (NUM_CHUNKS - 1) % 2
    store_e(NUM_CHUNKS - 1, bl).wait()
    store_o(NUM_CHUNKS - 1, bl).wait()


BB = 64
PAIRS_PER_BB = BB * SEQ // 2


def _tc_unpack_body(s_ref, o_ref):
    s3 = s_ref[...].reshape(BB, SEQ // 2, 2 * D_MODEL)
    o_ref[:, 0 : SEQ // 2, :] = s3[:, :, 0:D_MODEL]
    o_ref[:, SEQ // 2 : SEQ, :] = s3[:, :, D_MODEL : 2 * D_MODEL]


def _tc_unpack(stag):
    return pl.pallas_call(
        _tc_unpack_body,
        grid=(BATCH // BB,),
        in_specs=[pl.BlockSpec((PAIRS_PER_BB, 2 * D_MODEL), lambda i: (i, 0))],
        out_specs=pl.BlockSpec((BB, SEQ, D_MODEL), lambda i: (i, 0, 0)),
        out_shape=jax.ShapeDtypeStruct((BATCH, SEQ, D_MODEL), jnp.float32),
    )(stag)


def kernel(x, W):
    xi = x.astype(jnp.int32)
    idx_lo = xi[:, : SEQ // 2].reshape(-1)
    idx_hi = xi[:, SEQ // 2 :].reshape(-1)
    stag = _sc_gather(idx_lo, idx_hi, W)
    lo = stag[:, :D_MODEL].reshape(BATCH, SEQ // 2, D_MODEL)
    hi = stag[:, D_MODEL:].reshape(BATCH, SEQ // 2, D_MODEL)
    return jnp.concatenate([lo, hi], axis=1)


# speedup vs baseline: 2.4301x; 2.4301x over previous
import functools

import jax
import jax.numpy as jnp
from jax import lax
from jax.experimental import pallas as pl
from jax.experimental.pallas import tpu as pltpu
from jax.experimental.pallas import tpu_sc as plsc

D_MODEL = 64
BATCH = 4096
SEQ = 200
NUM_ROWS = BATCH * SEQ
NUM_PAIRS = NUM_ROWS // 2

NUM_CORES = 2
NUM_SUBCORES = 16
NUM_WORKERS = NUM_CORES * NUM_SUBCORES
PAIRS_PER_W = NUM_PAIRS // NUM_WORKERS
C2 = 320
NUM_CHUNKS = PAIRS_PER_W // C2

_MESH = plsc.VectorSubcoreMesh(core_axis_name="c", subcore_axis_name="s")


@functools.partial(
    pl.kernel,
    mesh=_MESH,
    out_type=jax.ShapeDtypeStruct((NUM_PAIRS, 2 * D_MODEL), jnp.float32),
    scratch_types=[
        pltpu.VMEM((PAIRS_PER_W,), jnp.int32),
        pltpu.VMEM((PAIRS_PER_W,), jnp.int32),
        pltpu.VMEM((C2, D_MODEL), jnp.float32),
        pltpu.VMEM((C2, D_MODEL), jnp.float32),
        pltpu.VMEM((C2, D_MODEL), jnp.float32),
        pltpu.VMEM((C2, D_MODEL), jnp.float32),
        pltpu.SemaphoreType.DMA,
        pltpu.SemaphoreType.DMA,
        pltpu.SemaphoreType.DMA,
        pltpu.SemaphoreType.DMA,
        pltpu.SemaphoreType.DMA,
        pltpu.SemaphoreType.DMA,
        pltpu.SemaphoreType.DMA,
        pltpu.SemaphoreType.DMA,
    ],
    compiler_params=pltpu.CompilerParams(use_tc_tiling_on_sc=False),
)
def _sc_gather(
    idx_e_hbm, idx_o_hbm, table_hbm, stag_hbm,
    idxe_v, idxo_v,
    rbe0, rbe1, rbo0, rbo1,
    ge0, ge1, go0, go1, se0, se1, so0, so1,
):
    wid = lax.axis_index("s") * NUM_CORES + lax.axis_index("c")
    base = wid * PAIRS_PER_W
    pltpu.sync_copy(idx_e_hbm.at[pl.ds(base, PAIRS_PER_W)], idxe_v)
    pltpu.sync_copy(idx_o_hbm.at[pl.ds(base, PAIRS_PER_W)], idxo_v)

    rbe = (rbe0, rbe1)
    rbo = (rbo0, rbo1)
    ge = (ge0, ge1)
    go = (go0, go1)
    se = (se0, se1)
    so = (so0, so1)

    def gather_e(i, b):
        return pltpu.make_async_copy(
            table_hbm.at[idxe_v.at[pl.ds(i * C2, C2)]], rbe[b], ge[b]
        )

    def gather_o(i, b):
        return pltpu.make_async_copy(
            table_hbm.at[idxo_v.at[pl.ds(i * C2, C2)]], rbo[b], go[b]
        )

    def store_e(i, b):
        return pltpu.make_async_copy(
            rbe[b],
            stag_hbm.at[pl.ds(base + i * C2, C2), pl.ds(0, D_MODEL)],
            se[b],
        )

    def store_o(i, b):
        return pltpu.make_async_copy(
            rbo[b],
            stag_hbm.at[pl.ds(base + i * C2, C2), pl.ds(D_MODEL, D_MODEL)],
            so[b],
        )

    gather_e(0, 0).start()
    gather_o(0, 0).start()
    for i in range(NUM_CHUNKS):
        b = i % 2
        gather_e(i, b).wait()
        gather_o(i, b).wait()
        if i >= 1:
            store_e(i - 1, 1 - b).wait()
            store_o(i - 1, 1 - b).wait()
        if i + 1 < NUM_CHUNKS:
            gather_e(i + 1, 1 - b).start()
            gather_o(i + 1, 1 - b).start()
        store_e(i, b).start()
        store_o(i, b).start()
    bl = (NUM_CHUNKS - 1) % 2
    store_e(NUM_CHUNKS - 1, bl).wait()
    store_o(NUM_CHUNKS - 1, bl).wait()


BB = 32
PAIRS_PER_BB = BB * SEQ // 2


def _tc_unpack_body(s_ref, o_ref):
    s3 = s_ref[...].reshape(BB, SEQ // 2, 2 * D_MODEL)
    o_ref[:, 0 : SEQ // 2, :] = s3[:, :, 0:D_MODEL]
    o_ref[:, SEQ // 2 : SEQ, :] = s3[:, :, D_MODEL : 2 * D_MODEL]


def _tc_unpack(stag):
    return pl.pallas_call(
        _tc_unpack_body,
        grid=(BATCH // BB,),
        in_specs=[pl.BlockSpec((PAIRS_PER_BB, 2 * D_MODEL), lambda i: (i, 0))],
        out_specs=pl.BlockSpec((BB, SEQ, D_MODEL), lambda i: (i, 0, 0)),
        out_shape=jax.ShapeDtypeStruct((BATCH, SEQ, D_MODEL), jnp.float32),
    )(stag)


def kernel(x, W):
    xi = x.astype(jnp.int32)
    idx_lo = xi[:, : SEQ // 2].reshape(-1)
    idx_hi = xi[:, SEQ // 2 :].reshape(-1)
    stag = _sc_gather(idx_lo, idx_hi, W)
    return _tc_unpack(stag)


# speedup vs baseline: 2.5077x; 1.0319x over previous
import functools

import jax
import jax.numpy as jnp
from jax import lax
from jax.experimental import pallas as pl
from jax.experimental.pallas import tpu as pltpu
from jax.experimental.pallas import tpu_sc as plsc

D_MODEL = 64
BATCH = 4096
SEQ = 200
NUM_ROWS = BATCH * SEQ
NUM_PAIRS = NUM_ROWS // 2

NUM_CORES = 2
NUM_SUBCORES = 16
NUM_WORKERS = NUM_CORES * NUM_SUBCORES
PAIRS_PER_W = NUM_PAIRS // NUM_WORKERS
C2 = 320
NUM_CHUNKS = PAIRS_PER_W // C2

_MESH = plsc.VectorSubcoreMesh(core_axis_name="c", subcore_axis_name="s")


@functools.partial(
    pl.kernel,
    mesh=_MESH,
    out_type=jax.ShapeDtypeStruct((NUM_PAIRS, 2 * D_MODEL), jnp.float32),
    scratch_types=[
        pltpu.VMEM((PAIRS_PER_W,), jnp.int32),
        pltpu.VMEM((PAIRS_PER_W,), jnp.int32),
        pltpu.VMEM((C2, D_MODEL), jnp.float32),
        pltpu.VMEM((C2, D_MODEL), jnp.float32),
        pltpu.VMEM((C2, D_MODEL), jnp.float32),
        pltpu.VMEM((C2, D_MODEL), jnp.float32),
        pltpu.SemaphoreType.DMA,
        pltpu.SemaphoreType.DMA,
        pltpu.SemaphoreType.DMA,
        pltpu.SemaphoreType.DMA,
        pltpu.SemaphoreType.DMA,
        pltpu.SemaphoreType.DMA,
        pltpu.SemaphoreType.DMA,
        pltpu.SemaphoreType.DMA,
    ],
    compiler_params=pltpu.CompilerParams(use_tc_tiling_on_sc=False),
)
def _sc_gather(
    idx_e_hbm, idx_o_hbm, table_hbm, stag_hbm,
    idxe_v, idxo_v,
    rbe0, rbe1, rbo0, rbo1,
    ge0, ge1, go0, go1, se0, se1, so0, so1,
):
    wid = lax.axis_index("s") * NUM_CORES + lax.axis_index("c")
    base = wid * PAIRS_PER_W
    pltpu.sync_copy(idx_e_hbm.at[pl.ds(base, PAIRS_PER_W)], idxe_v)
    pltpu.sync_copy(idx_o_hbm.at[pl.ds(base, PAIRS_PER_W)], idxo_v)

    rbe = (rbe0, rbe1)
    rbo = (rbo0, rbo1)
    ge = (ge0, ge1)
    go = (go0, go1)
    se = (se0, se1)
    so = (so0, so1)

    def gather_e(i, b):
        return pltpu.make_async_copy(
            table_hbm.at[idxe_v.at[pl.ds(i * C2, C2)]], rbe[b], ge[b]
        )

    def gather_o(i, b):
        return pltpu.make_async_copy(
            table_hbm.at[idxo_v.at[pl.ds(i * C2, C2)]], rbo[b], go[b]
        )

    def store_e(i, b):
        return pltpu.make_async_copy(
            rbe[b],
            stag_hbm.at[pl.ds(base + i * C2, C2), pl.ds(0, D_MODEL)],
            se[b],
        )

    def store_o(i, b):
        return pltpu.make_async_copy(
            rbo[b],
            stag_hbm.at[pl.ds(base + i * C2, C2), pl.ds(D_MODEL, D_MODEL)],
            so[b],
        )

    gather_e(0, 0).start()
    gather_o(0, 0).start()
    for i in range(NUM_CHUNKS):
        b = i % 2
        gather_e(i, b).wait()
        gather_o(i, b).wait()
        if i >= 1:
            store_e(i - 1, 1 - b).wait()
            store_o(i - 1, 1 - b).wait()
        if i + 1 < NUM_CHUNKS:
            gather_e(i + 1, 1 - b).start()
            gather_o(i + 1, 1 - b).start()
        store_e(i, b).start()
        store_o(i, b).start()
    bl = (NUM_CHUNKS - 1) % 2
    store_e(NUM_CHUNKS - 1, bl).wait()
    store_o(NUM_CHUNKS - 1, bl).wait()


BB = 128
PAIRS_PER_BB = BB * SEQ // 2


def _tc_unpack_body(s_ref, o_ref):
    s3 = s_ref[...].reshape(BB, SEQ // 2, 2 * D_MODEL)
    o_ref[:, 0 : SEQ // 2, :] = s3[:, :, 0:D_MODEL]
    o_ref[:, SEQ // 2 : SEQ, :] = s3[:, :, D_MODEL : 2 * D_MODEL]


def _tc_unpack(stag):
    return pl.pallas_call(
        _tc_unpack_body,
        grid=(BATCH // BB,),
        in_specs=[pl.BlockSpec((PAIRS_PER_BB, 2 * D_MODEL), lambda i: (i, 0))],
        out_specs=pl.BlockSpec((BB, SEQ, D_MODEL), lambda i: (i, 0, 0)),
        out_shape=jax.ShapeDtypeStruct((BATCH, SEQ, D_MODEL), jnp.float32),
    )(stag)


def kernel(x, W):
    xi = x.astype(jnp.int32)
    idx_lo = xi[:, : SEQ // 2].reshape(-1)
    idx_hi = xi[:, SEQ // 2 :].reshape(-1)
    stag = _sc_gather(idx_lo, idx_hi, W)
    return _tc_unpack(stag)


# speedup vs baseline: 2.5124x; 1.0019x over previous
import functools

import jax
import jax.numpy as jnp
from jax import lax
from jax.experimental import pallas as pl
from jax.experimental.pallas import tpu as pltpu
from jax.experimental.pallas import tpu_sc as plsc

D_MODEL = 64
BATCH = 4096
SEQ = 200
NUM_ROWS = BATCH * SEQ
NUM_PAIRS = NUM_ROWS // 2

NUM_CORES = 2
NUM_SUBCORES = 16
NUM_WORKERS = NUM_CORES * NUM_SUBCORES
PAIRS_PER_W = NUM_PAIRS // NUM_WORKERS
C2 = 400
NUM_CHUNKS = PAIRS_PER_W // C2

_MESH = plsc.VectorSubcoreMesh(core_axis_name="c", subcore_axis_name="s")


@functools.partial(
    pl.kernel,
    mesh=_MESH,
    out_type=jax.ShapeDtypeStruct((NUM_PAIRS, 2 * D_MODEL), jnp.float32),
    scratch_types=[
        pltpu.VMEM((PAIRS_PER_W,), jnp.int32),
        pltpu.VMEM((PAIRS_PER_W,), jnp.int32),
        pltpu.VMEM((C2, D_MODEL), jnp.float32),
        pltpu.VMEM((C2, D_MODEL), jnp.float32),
        pltpu.VMEM((C2, D_MODEL), jnp.float32),
        pltpu.VMEM((C2, D_MODEL), jnp.float32),
        pltpu.SemaphoreType.DMA,
        pltpu.SemaphoreType.DMA,
        pltpu.SemaphoreType.DMA,
        pltpu.SemaphoreType.DMA,
        pltpu.SemaphoreType.DMA,
        pltpu.SemaphoreType.DMA,
        pltpu.SemaphoreType.DMA,
        pltpu.SemaphoreType.DMA,
    ],
    compiler_params=pltpu.CompilerParams(use_tc_tiling_on_sc=False),
)
def _sc_gather(
    idx_e_hbm, idx_o_hbm, table_hbm, stag_hbm,
    idxe_v, idxo_v,
    rbe0, rbe1, rbo0, rbo1,
    ge0, ge1, go0, go1, se0, se1, so0, so1,
):
    wid = lax.axis_index("s") * NUM_CORES + lax.axis_index("c")
    base = wid * PAIRS_PER_W
    pltpu.sync_copy(idx_e_hbm.at[pl.ds(base, PAIRS_PER_W)], idxe_v)
    pltpu.sync_copy(idx_o_hbm.at[pl.ds(base, PAIRS_PER_W)], idxo_v)

    rbe = (rbe0, rbe1)
    rbo = (rbo0, rbo1)
    ge = (ge0, ge1)
    go = (go0, go1)
    se = (se0, se1)
    so = (so0, so1)

    def gather_e(i, b):
        return pltpu.make_async_copy(
            table_hbm.at[idxe_v.at[pl.ds(i * C2, C2)]], rbe[b], ge[b]
        )

    def gather_o(i, b):
        return pltpu.make_async_copy(
            table_hbm.at[idxo_v.at[pl.ds(i * C2, C2)]], rbo[b], go[b]
        )

    def store_e(i, b):
        return pltpu.make_async_copy(
            rbe[b],
            stag_hbm.at[pl.ds(base + i * C2, C2), pl.ds(0, D_MODEL)],
            se[b],
        )

    def store_o(i, b):
        return pltpu.make_async_copy(
            rbo[b],
            stag_hbm.at[pl.ds(base + i * C2, C2), pl.ds(D_MODEL, D_MODEL)],
            so[b],
        )

    gather_e(0, 0).start()
    gather_o(0, 0).start()
    for i in range(NUM_CHUNKS):
        b = i % 2
        gather_e(i, b).wait()
        gather_o(i, b).wait()
        if i >= 1:
            store_e(i - 1, 1 - b).wait()
            store_o(i - 1, 1 - b).wait()
        if i + 1 < NUM_CHUNKS:
            gather_e(i + 1, 1 - b).start()
            gather_o(i + 1, 1 - b).start()
        store_e(i, b).start()
        store_o(i, b).start()
    bl = (NUM_CHUNKS - 1) % 2
    store_e(NUM_CHUNKS - 1, bl).wait()
    store_o(NUM_CHUNKS - 1, bl).wait()


BB = 128
PAIRS_PER_BB = BB * SEQ // 2


def _tc_unpack_body(s_ref, o_ref):
    s3 = s_ref[...].reshape(BB, SEQ // 2, 2 * D_MODEL)
    o_ref[:, 0 : SEQ // 2, :] = s3[:, :, 0:D_MODEL]
    o_ref[:, SEQ // 2 : SEQ, :] = s3[:, :, D_MODEL : 2 * D_MODEL]


def _tc_unpack(stag):
    return pl.pallas_call(
        _tc_unpack_body,
        grid=(BATCH // BB,),
        in_specs=[pl.BlockSpec((PAIRS_PER_BB, 2 * D_MODEL), lambda i: (i, 0))],
        out_specs=pl.BlockSpec((BB, SEQ, D_MODEL), lambda i: (i, 0, 0)),
        out_shape=jax.ShapeDtypeStruct((BATCH, SEQ, D_MODEL), jnp.float32),
    )(stag)


def kernel(x, W):
    xi = x.astype(jnp.int32)
    idx_lo = xi[:, : SEQ // 2].reshape(-1)
    idx_hi = xi[:, SEQ // 2 :].reshape(-1)
    stag = _sc_gather(idx_lo, idx_hi, W)
    return _tc_unpack(stag)
